# baseline (speedup 1.0000x reference)
import functools

import jax
import jax.numpy as jnp
from jax import lax
from jax.experimental import pallas as pl
from jax.experimental.pallas import tpu as pltpu
from jax.experimental.pallas import tpu_sc as plsc

N_NODES = 10000
N_EDGES = 320000
D = 128
DH = D // 2

NC = 2
NS = 16
B = 128
NBE = 160
STG = 40
NPH = NBE // STG
E_PAD = NS * NBE * B
NBUF = 4
N_PAD = 10240
ROWS_PER_TILE = N_PAD // NS
NCHUNK = ROWS_PER_TILE // B
PAD_DST = N_NODES

_mesh = lambda: plsc.VectorSubcoreMesh(core_axis_name="c", subcore_axis_name="s")


def _edge_body(src_hbm, dst_hbm, x_hbm, out_hbm, deg_out,
               src_v, dst_v, rows_v, ones_v, deg_v, dinv_v, sems,
               y_sh, agg_sh, deg_sh):
    c = lax.axis_index("c")
    s = lax.axis_index("s")

    def _o(i, _):
        ones_v[pl.ds(i * 16, 16)] = jnp.ones((16,), jnp.float32)
        return 0
    lax.fori_loop(0, B // 16, _o, 0)

    def _d1(i, _):
        deg_v[pl.ds(i * 16, 16)] = jnp.ones((16,), jnp.float32)
        return 0
    lax.fori_loop(0, ROWS_PER_TILE // 16, _d1, 0)
    pltpu.sync_copy(deg_v, deg_sh.at[pl.ds(s * ROWS_PER_TILE, ROWS_PER_TILE)])

    def _z(i, _):
        rows_v[0, i // 4, pl.ds((i % 4) * 16, 16)] = jnp.zeros((16,),
                                                               jnp.float32)
        return 0
    lax.fori_loop(0, B * (DH // 16), _z, 0)
    for k in range(NCHUNK):
        pltpu.sync_copy(rows_v.at[0],
                        agg_sh.at[pl.ds(s * ROWS_PER_TILE + k * B, B)])

    plsc.subcore_barrier()

    for p in range(NPH):
        pltpu.sync_copy(dst_hbm.at[s, pl.ds(p * STG, STG)], dst_v)

        def _acc(j, _):
            pltpu.sync_copy(ones_v, deg_sh.at[dst_v.at[j]], add=True)
            return 0
        lax.fori_loop(0, STG, _acc, 0)

    plsc.subcore_barrier()

    pltpu.sync_copy(deg_sh.at[pl.ds(s * ROWS_PER_TILE, ROWS_PER_TILE)], deg_v)

    def _rs(i, _):
        dg = deg_v[pl.ds(i * 16, 16)]
        iz = jnp.int32(0x5F3759DF) - lax.shift_right_logical(
            plsc.bitcast(dg, jnp.int32), jnp.int32(1))
        z = plsc.bitcast(iz, jnp.float32)
        z = z * (1.5 - 0.5 * dg * z * z)
        z = z * (1.5 - 0.5 * dg * z * z)
        z = z * (1.5 - 0.5 * dg * z * z)
        dinv_v[pl.ds(i * 16, 16)] = z
        return 0
    lax.fori_loop(0, ROWS_PER_TILE // 16, _rs, 0)

    for k in range(NCHUNK):
        pltpu.sync_copy(
            x_hbm.at[c, pl.ds(s * ROWS_PER_TILE + k * B, B)], rows_v.at[0])

        def _sc(rr, _):
            dvs = dinv_v[pl.ds(k * B + rr * 16, 16)]
            for lane in range(16):
                r = rr * 16 + lane
                dv = dvs[lane]
                for cc in range(DH // 16):
                    rows_v[0, r, pl.ds(cc * 16, 16)] = (
                        rows_v[0, r, pl.ds(cc * 16, 16)] * dv)
            return 0
        lax.fori_loop(0, B // 16, _sc, 0)
        pltpu.sync_copy(rows_v.at[0],
                        y_sh.at[pl.ds(s * ROWS_PER_TILE + k * B, B)])

    plsc.subcore_barrier()

    def _gather(j, b):
        pltpu.async_copy(y_sh.at[src_v.at[j]], rows_v.at[b], sems.at[b])

    def _scatter(j, b):
        pltpu.make_async_copy(y_sh.at[src_v.at[j]], rows_v.at[b],
                              sems.at[b]).wait()
        pltpu.sync_copy(rows_v.at[b], agg_sh.at[dst_v.at[j]], add=True)

    for p in range(NPH):
        pltpu.sync_copy(src_hbm.at[s, pl.ds(p * STG, STG)], src_v)
        pltpu.sync_copy(dst_hbm.at[s, pl.ds(p * STG, STG)], dst_v)
        for b in range(NBUF):
            _gather(b, b)

        def _grp(g, _):
            for b in range(NBUF):
                j = g * NBUF + b
                _scatter(j, b)
                _gather(j + NBUF, b)
            return 0
        lax.fori_loop(0, STG // NBUF - 1, _grp, 0)
        for b in range(NBUF):
            _scatter(STG - NBUF + b, b)

    plsc.subcore_barrier()
    pltpu.sync_copy(agg_sh.at[pl.ds(s * ROWS_PER_TILE, ROWS_PER_TILE)],
                    out_hbm.at[c, pl.ds(s * ROWS_PER_TILE, ROWS_PER_TILE)])
    pltpu.sync_copy(deg_sh.at[pl.ds(s * ROWS_PER_TILE, ROWS_PER_TILE)],
                    deg_out.at[c, pl.ds(s * ROWS_PER_TILE, ROWS_PER_TILE)])


def _edge_call(src3, dst3, xh):
    f = functools.partial(
        pl.kernel,
        out_type=(jax.ShapeDtypeStruct((NC, N_PAD, DH), jnp.float32),
                  jax.ShapeDtypeStruct((NC, N_PAD), jnp.float32)),
        mesh=_mesh(),
        scratch_types=[
            pltpu.VMEM((STG, B), jnp.int32),
            pltpu.VMEM((STG, B), jnp.int32),
            pltpu.VMEM((NBUF, B, DH), jnp.float32),
            pltpu.VMEM((B,), jnp.float32),
            pltpu.VMEM((ROWS_PER_TILE,), jnp.float32),
            pltpu.VMEM((ROWS_PER_TILE + 16,), jnp.float32),
            pltpu.SemaphoreType.DMA((NBUF,)),
            pltpu.VMEM_SHARED((N_PAD, DH), jnp.float32),
            pltpu.VMEM_SHARED((N_PAD, DH), jnp.float32),
            pltpu.VMEM_SHARED((N_PAD,), jnp.float32),
        ],
        compiler_params=pltpu.CompilerParams(use_tc_tiling_on_sc=False,
                                             needs_layout_passes=False),
    )(_edge_body)
    return f(src3, dst3, xh)


RB = 2048


def _fin_body(p_ref, x_ref, deg_ref, w_ref, b_ref, g_ref, be_ref, o_ref):
    i = pl.program_id(0)
    st = pl.multiple_of(i * RB, 128)
    dinv = lax.rsqrt(deg_ref[0, pl.ds(st, RB)])
    p = jnp.concatenate([p_ref[0], p_ref[1]], axis=-1)
    pre = (p + x_ref[...] * dinv[:, None]) * dinv[:, None]
    t = jnp.dot(pre, w_ref[...], preferred_element_type=jnp.float32,
                precision=lax.Precision.HIGHEST)
    t = t + b_ref[...]
    mean = jnp.mean(t, axis=-1, keepdims=True)
    var = jnp.mean((t - mean) ** 2, axis=-1, keepdims=True)
    o_ref[...] = (t - mean) / jnp.sqrt(var + 1e-5) * g_ref[...] + be_ref[...]


def _fin_call(P, x_p, degp, W, b, gamma, beta):
    return pl.pallas_call(
        _fin_body,
        grid=(N_PAD // RB,),
        in_specs=[
            pl.BlockSpec((NC, RB, DH), lambda i: (0, i, 0)),
            pl.BlockSpec((RB, D), lambda i: (i, 0)),
            pl.BlockSpec((NC, N_PAD), lambda i: (0, 0)),
            pl.BlockSpec((D, D), lambda i: (0, 0)),
            pl.BlockSpec((D,), lambda i: (0,)),
            pl.BlockSpec((D,), lambda i: (0,)),
            pl.BlockSpec((D,), lambda i: (0,)),
        ],
        out_specs=pl.BlockSpec((RB, D), lambda i: (i, 0)),
        out_shape=jax.ShapeDtypeStruct((N_NODES, D), jnp.float32),
    )(P, x_p, degp, W, b, gamma, beta)


def kernel(x, edge_index, W, b, gamma, beta):
    ei = edge_index.astype(jnp.int32)
    pad = E_PAD - N_EDGES
    src3 = jnp.concatenate(
        [ei[0], jnp.zeros((pad,), jnp.int32)]).reshape(NS, NBE, B)
    dst3 = jnp.concatenate(
        [ei[1], jnp.full((pad,), PAD_DST, jnp.int32)]).reshape(NS, NBE, B)
    x_p = jnp.concatenate(
        [x, jnp.zeros((N_PAD - N_NODES, D), jnp.float32)])
    xh = jnp.stack([x_p[:, :DH], x_p[:, DH:]])
    P, degp = _edge_call(src3, dst3, xh)
    return _fin_call(P, x_p, degp, W, b, gamma, beta)


# speedup vs baseline: 1.1135x; 1.1135x over previous
import functools

import jax
import jax.numpy as jnp
from jax import lax
from jax.experimental import pallas as pl
from jax.experimental.pallas import tpu as pltpu
from jax.experimental.pallas import tpu_sc as plsc

N_NODES = 10000
N_EDGES = 320000
D = 128
DH = D // 2

NC = 2
NS = 16
B = 125
NBE = 160
NBD = NBE // NC
STG = 40
NBUF = 4
N_PAD = 10240
ROWS_PER_TILE = N_PAD // NS
DEG_PAD = 10240
DEG_PER_TILE = DEG_PAD // NS

_mesh = lambda: plsc.VectorSubcoreMesh(core_axis_name="c", subcore_axis_name="s")


def _deg_body(dst_hbm, out_hbm, idx_v, ones_v, zed_v, deg_sh):
    c = lax.axis_index("c")
    s = lax.axis_index("s")

    def _z(i, _):
        zed_v[pl.ds(i * 16, 16)] = jnp.zeros((16,), jnp.float32)
        return 0
    lax.fori_loop(0, DEG_PER_TILE // 16, _z, 0)
    pltpu.sync_copy(zed_v, deg_sh.at[pl.ds(s * DEG_PER_TILE, DEG_PER_TILE)])

    def _o(i, _):
        ones_v[pl.ds(i * 16, 16)] = jnp.ones((16,), jnp.float32)
        return 0
    lax.fori_loop(0, 128 // 16, _o, 0)

    pltpu.sync_copy(dst_hbm.at[s, pl.ds(c * NBD, NBD)], idx_v)

    plsc.subcore_barrier()

    def _acc(j, _):
        pltpu.sync_copy(ones_v.at[pl.ds(0, B)], deg_sh.at[idx_v.at[j]],
                        add=True)
        return 0
    lax.fori_loop(0, NBD, _acc, 0)

    plsc.subcore_barrier()
    pltpu.sync_copy(deg_sh.at[pl.ds(s * DEG_PER_TILE, DEG_PER_TILE)],
                    out_hbm.at[c, pl.ds(s * DEG_PER_TILE, DEG_PER_TILE)])


def _deg_call(dst3):
    f = functools.partial(
        pl.kernel,
        out_type=jax.ShapeDtypeStruct((NC, DEG_PAD), jnp.float32),
        mesh=_mesh(),
        scratch_types=[
            pltpu.VMEM((NBD, B), jnp.int32),
            pltpu.VMEM((128,), jnp.float32),
            pltpu.VMEM((DEG_PER_TILE,), jnp.float32),
            pltpu.VMEM_SHARED((DEG_PAD,), jnp.float32),
        ],
    )(_deg_body)
    return f(dst3)


def _edge_body(src_hbm, dst_hbm, y_hbm, out_hbm,
               src_v, dst_v, rows_v, sems, y_sh, agg_sh):
    c = lax.axis_index("c")
    s = lax.axis_index("s")

    pltpu.sync_copy(y_hbm.at[c, pl.ds(s * ROWS_PER_TILE, ROWS_PER_TILE)],
                    y_sh.at[pl.ds(s * ROWS_PER_TILE, ROWS_PER_TILE)])

    def _z(i, _):
        rows_v[0, i // 4, pl.ds((i % 4) * 16, 16)] = jnp.zeros((16,),
                                                               jnp.float32)
        return 0
    lax.fori_loop(0, B * (DH // 16), _z, 0)
    for k in range(ROWS_PER_TILE // B):
        pltpu.sync_copy(rows_v.at[0],
                        agg_sh.at[pl.ds(s * ROWS_PER_TILE + k * B, B)])
    rem = ROWS_PER_TILE - (ROWS_PER_TILE // B) * B
    pltpu.sync_copy(
        rows_v.at[0, pl.ds(0, rem)],
        agg_sh.at[pl.ds(s * ROWS_PER_TILE + (ROWS_PER_TILE // B) * B, rem)])

    plsc.subcore_barrier()

    def _gather(j, b):
        pltpu.async_copy(y_sh.at[src_v.at[j]], rows_v.at[b], sems.at[b])

    def _scatter(j, b):
        pltpu.make_async_copy(y_sh.at[src_v.at[j]], rows_v.at[b],
                              sems.at[b]).wait()
        pltpu.sync_copy(rows_v.at[b], agg_sh.at[dst_v.at[j]], add=True)

    for p in range(NBE // STG):
        pltpu.sync_copy(src_hbm.at[s, pl.ds(p * STG, STG)], src_v)
        pltpu.sync_copy(dst_hbm.at[s, pl.ds(p * STG, STG)], dst_v)
        for b in range(NBUF):
            _gather(b, b)

        def _grp(g, _):
            for b in range(NBUF):
                j = g * NBUF + b
                _scatter(j, b)
                _gather(j + NBUF, b)
            return 0
        lax.fori_loop(0, STG // NBUF - 1, _grp, 0)
        for b in range(NBUF):
            _scatter(STG - NBUF + b, b)

    plsc.subcore_barrier()
    pltpu.sync_copy(agg_sh.at[pl.ds(s * ROWS_PER_TILE, ROWS_PER_TILE)],
                    out_hbm.at[c, pl.ds(s * ROWS_PER_TILE, ROWS_PER_TILE)])


def _edge_call(src3, dst3, y):
    f = functools.partial(
        pl.kernel,
        out_type=jax.ShapeDtypeStruct((NC, N_PAD, DH), jnp.float32),
        mesh=_mesh(),
        scratch_types=[
            pltpu.VMEM((STG, B), jnp.int32),
            pltpu.VMEM((STG, B), jnp.int32),
            pltpu.VMEM((NBUF, B, DH), jnp.float32),
            pltpu.SemaphoreType.DMA((NBUF,)),
            pltpu.VMEM_SHARED((N_PAD, DH), jnp.float32),
            pltpu.VMEM_SHARED((N_PAD, DH), jnp.float32),
        ],
        compiler_params=pltpu.CompilerParams(use_tc_tiling_on_sc=False),
    )(_edge_body)
    return f(src3, dst3, y)


RB = 2048


def _deg_slice(degp_ref, i):
    st = pl.multiple_of(i * RB, 128)
    deg = degp_ref[0, pl.ds(st, RB)] + degp_ref[1, pl.ds(st, RB)] + 1.0
    return lax.rsqrt(deg)


def _mm_body(x_ref, w_ref, degp_ref, y_ref):
    dinv = _deg_slice(degp_ref, pl.program_id(0))
    xw = jnp.dot(x_ref[...], w_ref[...], preferred_element_type=jnp.float32)
    y = xw * dinv[:, None]
    y_ref[0] = y[:, :DH]
    y_ref[1] = y[:, DH:]


def _mm_call(x_p, W, degp):
    return pl.pallas_call(
        _mm_body,
        grid=(N_PAD // RB,),
        in_specs=[
            pl.BlockSpec((RB, D), lambda i: (i, 0)),
            pl.BlockSpec((D, D), lambda i: (0, 0)),
            pl.BlockSpec((NC, DEG_PAD), lambda i: (0, 0)),
        ],
        out_specs=pl.BlockSpec((NC, RB, DH), lambda i: (0, i, 0)),
        out_shape=jax.ShapeDtypeStruct((NC, N_PAD, DH), jnp.float32),
    )(x_p, W, degp)


def _fin_body(p_ref, y_ref, degp_ref, b_ref, g_ref, be_ref, o_ref):
    dinv = _deg_slice(degp_ref, pl.program_id(0))
    p = jnp.concatenate([p_ref[0], p_ref[1]], axis=-1)
    y = jnp.concatenate([y_ref[0], y_ref[1]], axis=-1)
    t = (p + y) * dinv[:, None] + b_ref[...]
    mean = jnp.mean(t, axis=-1, keepdims=True)
    var = jnp.mean((t - mean) ** 2, axis=-1, keepdims=True)
    o_ref[...] = (t - mean) / jnp.sqrt(var + 1e-5) * g_ref[...] + be_ref[...]


def _fin_call(P, y, degp, b, gamma, beta):
    return pl.pallas_call(
        _fin_body,
        grid=(N_PAD // RB,),
        in_specs=[
            pl.BlockSpec((NC, RB, DH), lambda i: (0, i, 0)),
            pl.BlockSpec((NC, RB, DH), lambda i: (0, i, 0)),
            pl.BlockSpec((NC, DEG_PAD), lambda i: (0, 0)),
            pl.BlockSpec((D,), lambda i: (0,)),
            pl.BlockSpec((D,), lambda i: (0,)),
            pl.BlockSpec((D,), lambda i: (0,)),
        ],
        out_specs=pl.BlockSpec((RB, D), lambda i: (i, 0)),
        out_shape=jax.ShapeDtypeStruct((N_NODES, D), jnp.float32),
    )(P, y, degp, b, gamma, beta)


def kernel(x, edge_index, W, b, gamma, beta):
    ei = edge_index.astype(jnp.int32)
    src3 = ei[0].reshape(NS, NBE, B)
    dst3 = ei[1].reshape(NS, NBE, B)
    x_p = jnp.concatenate(
        [x, jnp.zeros((N_PAD - N_NODES, D), jnp.float32)])
    degp = _deg_call(dst3)
    y = _mm_call(x_p, W, degp)
    P = _edge_call(src3, dst3, y)
    return _fin_call(P, y, degp, b, gamma, beta)


# speedup vs baseline: 1.1521x; 1.0346x over previous
import functools

import jax
import jax.numpy as jnp
from jax import lax
from jax.experimental import pallas as pl
from jax.experimental.pallas import tpu as pltpu
from jax.experimental.pallas import tpu_sc as plsc

N_NODES = 10000
N_EDGES = 320000
D = 128
DH = D // 2

NC = 2
NS = 16
B = 125
NBE = 160
NBD = NBE // NC
STG = 80
NBUF = 2
N_PAD = 10240
ROWS_PER_TILE = N_PAD // NS
DEG_PAD = 10240
DEG_PER_TILE = DEG_PAD // NS

_mesh = lambda: plsc.VectorSubcoreMesh(core_axis_name="c", subcore_axis_name="s")


def _deg_body(dst_hbm, out_hbm, idx_v, ones_v, zed_v, deg_sh):
    c = lax.axis_index("c")
    s = lax.axis_index("s")

    def _z(i, _):
        zed_v[pl.ds(i * 16, 16)] = jnp.zeros((16,), jnp.float32)
        return 0
    lax.fori_loop(0, DEG_PER_TILE // 16, _z, 0)
    pltpu.sync_copy(zed_v, deg_sh.at[pl.ds(s * DEG_PER_TILE, DEG_PER_TILE)])

    def _o(i, _):
        ones_v[pl.ds(i * 16, 16)] = jnp.ones((16,), jnp.float32)
        return 0
    lax.fori_loop(0, 128 // 16, _o, 0)

    pltpu.sync_copy(dst_hbm.at[s, pl.ds(c * NBD, NBD)], idx_v)

    plsc.subcore_barrier()

    def _acc(j, _):
        pltpu.sync_copy(ones_v.at[pl.ds(0, B)], deg_sh.at[idx_v.at[j]],
                        add=True)
        return 0
    lax.fori_loop(0, NBD, _acc, 0)

    plsc.subcore_barrier()
    pltpu.sync_copy(deg_sh.at[pl.ds(s * DEG_PER_TILE, DEG_PER_TILE)],
                    out_hbm.at[c, pl.ds(s * DEG_PER_TILE, DEG_PER_TILE)])


def _deg_call(dst3):
    f = functools.partial(
        pl.kernel,
        out_type=jax.ShapeDtypeStruct((NC, DEG_PAD), jnp.float32),
        mesh=_mesh(),
        scratch_types=[
            pltpu.VMEM((NBD, B), jnp.int32),
            pltpu.VMEM((128,), jnp.float32),
            pltpu.VMEM((DEG_PER_TILE,), jnp.float32),
            pltpu.VMEM_SHARED((DEG_PAD,), jnp.float32),
        ],
    )(_deg_body)
    return f(dst3)


def _edge_body(src_hbm, dst_hbm, y_hbm, out_hbm,
               src_v, dst_v, rows_v, sems, y_sh, agg_sh):
    c = lax.axis_index("c")
    s = lax.axis_index("s")

    pltpu.sync_copy(y_hbm.at[c, pl.ds(s * ROWS_PER_TILE, ROWS_PER_TILE)],
                    y_sh.at[pl.ds(s * ROWS_PER_TILE, ROWS_PER_TILE)])

    def _z(i, _):
        rows_v[0, i // 4, pl.ds((i % 4) * 16, 16)] = jnp.zeros((16,),
                                                               jnp.float32)
        return 0
    lax.fori_loop(0, B * (DH // 16), _z, 0)
    for k in range(ROWS_PER_TILE // B):
        pltpu.sync_copy(rows_v.at[0],
                        agg_sh.at[pl.ds(s * ROWS_PER_TILE + k * B, B)])
    rem = ROWS_PER_TILE - (ROWS_PER_TILE // B) * B
    pltpu.sync_copy(
        rows_v.at[0, pl.ds(0, rem)],
        agg_sh.at[pl.ds(s * ROWS_PER_TILE + (ROWS_PER_TILE // B) * B, rem)])

    plsc.subcore_barrier()

    def _gather(j, b):
        pltpu.async_copy(y_sh.at[src_v.at[j]], rows_v.at[b], sems.at[b])

    def _scatter(j, b):
        pltpu.make_async_copy(y_sh.at[src_v.at[j]], rows_v.at[b],
                              sems.at[b]).wait()
        pltpu.sync_copy(rows_v.at[b], agg_sh.at[dst_v.at[j]], add=True)

    for p in range(NBE // STG):
        pltpu.sync_copy(src_hbm.at[s, pl.ds(p * STG, STG)], src_v)
        pltpu.sync_copy(dst_hbm.at[s, pl.ds(p * STG, STG)], dst_v)
        for b in range(NBUF):
            _gather(b, b)

        def _grp(g, _):
            for b in range(NBUF):
                j = g * NBUF + b
                _scatter(j, b)
                _gather(j + NBUF, b)
            return 0
        lax.fori_loop(0, STG // NBUF - 1, _grp, 0)
        for b in range(NBUF):
            _scatter(STG - NBUF + b, b)

    plsc.subcore_barrier()
    pltpu.sync_copy(agg_sh.at[pl.ds(s * ROWS_PER_TILE, ROWS_PER_TILE)],
                    out_hbm.at[c, pl.ds(s * ROWS_PER_TILE, ROWS_PER_TILE)])


def _edge_call(src3, dst3, y):
    f = functools.partial(
        pl.kernel,
        out_type=jax.ShapeDtypeStruct((NC, N_PAD, DH), jnp.float32),
        mesh=_mesh(),
        scratch_types=[
            pltpu.VMEM((STG, B), jnp.int32),
            pltpu.VMEM((STG, B), jnp.int32),
            pltpu.VMEM((NBUF, B, DH), jnp.float32),
            pltpu.SemaphoreType.DMA((NBUF,)),
            pltpu.VMEM_SHARED((N_PAD, DH), jnp.float32),
            pltpu.VMEM_SHARED((N_PAD, DH), jnp.float32),
        ],
        compiler_params=pltpu.CompilerParams(use_tc_tiling_on_sc=False),
    )(_edge_body)
    return f(src3, dst3, y)


RB = 2048


def _deg_slice(degp_ref, i):
    st = pl.multiple_of(i * RB, 128)
    deg = degp_ref[0, pl.ds(st, RB)] + degp_ref[1, pl.ds(st, RB)] + 1.0
    return lax.rsqrt(deg)


def _mm_body(x_ref, w_ref, degp_ref, y_ref):
    dinv = _deg_slice(degp_ref, pl.program_id(0))
    xw = jnp.dot(x_ref[...], w_ref[...], preferred_element_type=jnp.float32)
    y = xw * dinv[:, None]
    y_ref[0] = y[:, :DH]
    y_ref[1] = y[:, DH:]


def _mm_call(x_p, W, degp):
    return pl.pallas_call(
        _mm_body,
        grid=(N_PAD // RB,),
        in_specs=[
            pl.BlockSpec((RB, D), lambda i: (i, 0)),
            pl.BlockSpec((D, D), lambda i: (0, 0)),
            pl.BlockSpec((NC, DEG_PAD), lambda i: (0, 0)),
        ],
        out_specs=pl.BlockSpec((NC, RB, DH), lambda i: (0, i, 0)),
        out_shape=jax.ShapeDtypeStruct((NC, N_PAD, DH), jnp.float32),
    )(x_p, W, degp)


def _fin_body(p_ref, y_ref, degp_ref, b_ref, g_ref, be_ref, o_ref):
    dinv = _deg_slice(degp_ref, pl.program_id(0))
    p = jnp.concatenate([p_ref[0], p_ref[1]], axis=-1)
    y = jnp.concatenate([y_ref[0], y_ref[1]], axis=-1)
    t = (p + y) * dinv[:, None] + b_ref[...]
    mean = jnp.mean(t, axis=-1, keepdims=True)
    var = jnp.mean((t - mean) ** 2, axis=-1, keepdims=True)
    o_ref[...] = (t - mean) / jnp.sqrt(var + 1e-5) * g_ref[...] + be_ref[...]


def _fin_call(P, y, degp, b, gamma, beta):
    return pl.pallas_call(
        _fin_body,
        grid=(N_PAD // RB,),
        in_specs=[
            pl.BlockSpec((NC, RB, DH), lambda i: (0, i, 0)),
            pl.BlockSpec((NC, RB, DH), lambda i: (0, i, 0)),
            pl.BlockSpec((NC, DEG_PAD), lambda i: (0, 0)),
            pl.BlockSpec((D,), lambda i: (0,)),
            pl.BlockSpec((D,), lambda i: (0,)),
            pl.BlockSpec((D,), lambda i: (0,)),
        ],
        out_specs=pl.BlockSpec((RB, D), lambda i: (i, 0)),
        out_shape=jax.ShapeDtypeStruct((N_NODES, D), jnp.float32),
    )(P, y, degp, b, gamma, beta)


def kernel(x, edge_index, W, b, gamma, beta):
    ei = edge_index.astype(jnp.int32)
    src3 = ei[0].reshape(NS, NBE, B)
    dst3 = ei[1].reshape(NS, NBE, B)
    x_p = jnp.concatenate(
        [x, jnp.zeros((N_PAD - N_NODES, D), jnp.float32)])
    degp = _deg_call(dst3)
    y = _mm_call(x_p, W, degp)
    P = _edge_call(src3, dst3, y)
    return _fin_call(P, y, degp, b, gamma, beta)
